# Initial kernel scaffold; baseline (speedup 1.0000x reference)
import functools

import jax
import jax.numpy as jnp
from jax import lax
from jax.experimental import pallas as pl
from jax.experimental.pallas import tpu as pltpu
from jax.experimental.pallas import tpu_sc as plsc

N = 10000
NPAD = 10240
E = 160000
CHUNK = 128
NTILES = 32
QPT = 40
EPAD = NTILES * QPT * CHUNK
NC, NS = 2, 16
RPT = NPAD // NS
R_BLK = 1024
F0 = 64
F1 = 16


def _sc_mesh():
    return plsc.VectorSubcoreMesh(core_axis_name="c", subcore_axis_name="s")


def _deg_body(cols_hbm, out_hbm, accum, idxb, ones, zbuf):
    cid = lax.axis_index("c")
    sid = lax.axis_index("s")
    w = cid * NS + sid

    def fill_z(i, _):
        zbuf[pl.ds(i * 16, 16)] = jnp.zeros((16,), jnp.float32)
        return 0

    lax.fori_loop(0, RPT // 16, fill_z, 0)

    def fill_o(i, _):
        ones[pl.ds(i * 16, 16)] = jnp.ones((16,), jnp.float32)
        return 0

    lax.fori_loop(0, CHUNK // 16, fill_o, 0)

    for r in range(4):
        pltpu.sync_copy(zbuf, accum.at[pl.ds(sid * RPT, RPT)])
        plsc.subcore_barrier()
        pltpu.sync_copy(cols_hbm.at[r, w], idxb)

        def body(j, _):
            pltpu.sync_copy(ones, accum.at[idxb.at[j]], add=True)
            return 0

        lax.fori_loop(0, QPT, body, 0)
        plsc.subcore_barrier()
        pltpu.sync_copy(accum.at[pl.ds(sid * RPT, RPT)],
                        out_hbm.at[r, cid, pl.ds(sid * RPT, RPT)])
        plsc.subcore_barrier()


def _deg_call(cols_all):
    k = pl.kernel(
        _deg_body,
        out_type=jax.ShapeDtypeStruct((4, NC, NPAD), jnp.float32),
        mesh=_sc_mesh(),
        scratch_types=[
            pltpu.VMEM_SHARED((NPAD,), jnp.float32),
            pltpu.VMEM((QPT, CHUNK), jnp.int32),
            pltpu.VMEM((CHUNK,), jnp.float32),
            pltpu.VMEM((RPT,), jnp.float32),
        ],
    )
    return k(cols_all)


def _make_scatter_body(F):
    def body(rows_hbm, cols_hbm, t0, t1, t2, t3, out_hbm,
             accum, idxr, idxc, val, zb, sem):
        cid = lax.axis_index("c")
        sid = lax.axis_index("s")
        w = cid * NS + sid
        tables = [t0, t1, t2, t3]

        def fill_z(i, _):
            for kk in range(F // 16):
                zb[i, pl.ds(kk * 16, 16)] = jnp.zeros((16,), jnp.float32)
            return 0

        lax.fori_loop(0, CHUNK, fill_z, 0)

        for r in range(4):
            def zero_cp(j, _):
                pltpu.sync_copy(
                    zb, accum.at[pl.ds(sid * RPT + j * CHUNK, CHUNK)])
                return 0

            lax.fori_loop(0, RPT // CHUNK, zero_cp, 0)
            plsc.subcore_barrier()
            pltpu.sync_copy(rows_hbm.at[r, w], idxr)
            pltpu.sync_copy(cols_hbm.at[r, w], idxc)

            def chunk(j, _):
                pltpu.async_copy(tables[r].at[idxr.at[j]], val, sem).wait()
                pltpu.sync_copy(val, accum.at[idxc.at[j]], add=True)
                return 0

            lax.fori_loop(0, QPT, chunk, 0)
            plsc.subcore_barrier()
            pltpu.sync_copy(accum.at[pl.ds(sid * RPT, RPT)],
                            out_hbm.at[r, cid, pl.ds(sid * RPT, RPT)])
            plsc.subcore_barrier()

    return body


def _scatter_call(F, rows_all, cols_all, t_tt, t_aa, t_at, t_ta):
    k = pl.kernel(
        _make_scatter_body(F),
        out_type=jax.ShapeDtypeStruct((4, NC, NPAD, F), jnp.float32),
        mesh=_sc_mesh(),
        scratch_types=[
            pltpu.VMEM_SHARED((NPAD, F), jnp.float32),
            pltpu.VMEM((QPT, CHUNK), jnp.int32),
            pltpu.VMEM((QPT, CHUNK), jnp.int32),
            pltpu.VMEM((CHUNK, F), jnp.float32),
            pltpu.VMEM((CHUNK, F), jnp.float32),
            pltpu.SemaphoreType.DMA,
        ],
    )
    return k(rows_all, cols_all, t_tt, t_aa, t_at, t_ta)


def _dinvs(d):
    dinv_tt = lax.rsqrt(d[0] + d[1] + 1.0)
    dinv_aa = lax.rsqrt(d[2] + d[3] + 1.0)
    dinv_at = lax.rsqrt(d[4] + d[5] + 1.0)
    dinv_ta = lax.rsqrt(d[6] + d[7] + 1.0)
    return dinv_tt, dinv_aa, dinv_at, dinv_ta


def _tc0_body(xt_ref, xa_ref, wtx_ref, wad_ref, deg_ref,
              o_tt, o_aa, o_at, o_ta):
    dinv_tt, dinv_aa, dinv_at, dinv_ta = _dinvs(deg_ref[...])
    h_tx = jnp.dot(xt_ref[...], wtx_ref[...],
                   preferred_element_type=jnp.float32)
    h_ad = jnp.dot(xa_ref[...], wad_ref[...],
                   preferred_element_type=jnp.float32)
    o_tt[...] = h_tx[:, :F0] * dinv_tt[:, None]
    o_ta[...] = h_tx[:, F0:] * dinv_ta[:, None]
    o_aa[...] = h_ad[:, :F0] * dinv_aa[:, None]
    o_at[...] = h_ad[:, F0:] * dinv_at[:, None]


def _tc0_call(xt, xa, wtx, wad, deg8):
    grid = NPAD // R_BLK
    fo = jax.ShapeDtypeStruct((NPAD, F0), jnp.float32)
    return pl.pallas_call(
        _tc0_body,
        grid=(grid,),
        in_specs=[
            pl.BlockSpec((R_BLK, 128), lambda i: (i, 0)),
            pl.BlockSpec((R_BLK, 128), lambda i: (i, 0)),
            pl.BlockSpec((128, 128), lambda i: (0, 0)),
            pl.BlockSpec((128, 128), lambda i: (0, 0)),
            pl.BlockSpec((8, R_BLK), lambda i: (0, i)),
        ],
        out_specs=[pl.BlockSpec((R_BLK, F0), lambda i: (i, 0))] * 4,
        out_shape=[fo, fo, fo, fo],
    )(xt, xa, wtx, wad, deg8)


def _tc1_body(s_ref, hs_tt, hs_aa, hs_at, hs_ta, deg_ref, b0_ref,
              w1tx_ref, w1ad_ref, o_tt, o_aa, o_at, o_ta):
    dinv_tt, dinv_aa, dinv_at, dinv_ta = _dinvs(deg_ref[...])
    s = s_ref[...]
    b0 = b0_ref[...]
    tx1 = (dinv_tt[:, None] * (s[0] + s[1] + hs_tt[...]) + b0[0][None, :]
           + dinv_at[:, None] * (s[4] + s[5] + hs_at[...]) + b0[2][None, :])
    ad1 = (dinv_aa[:, None] * (s[2] + s[3] + hs_aa[...]) + b0[1][None, :]
           + dinv_ta[:, None] * (s[6] + s[7] + hs_ta[...]) + b0[3][None, :])
    tx1 = jnp.maximum(tx1, 0.0)
    ad1 = jnp.maximum(ad1, 0.0)
    h1tx = jnp.dot(tx1, w1tx_ref[...], preferred_element_type=jnp.float32)
    h1ad = jnp.dot(ad1, w1ad_ref[...], preferred_element_type=jnp.float32)
    o_tt[...] = h1tx[:, :F1] * dinv_tt[:, None]
    o_ta[...] = h1tx[:, F1:] * dinv_ta[:, None]
    o_aa[...] = h1ad[:, :F1] * dinv_aa[:, None]
    o_at[...] = h1ad[:, F1:] * dinv_at[:, None]


def _tc1_call(s0, hs_tt, hs_aa, hs_at, hs_ta, deg8, b0, w1tx, w1ad):
    grid = NPAD // R_BLK
    fo = jax.ShapeDtypeStruct((NPAD, F1), jnp.float32)
    hs_spec = pl.BlockSpec((R_BLK, F0), lambda i: (i, 0))
    return pl.pallas_call(
        _tc1_body,
        grid=(grid,),
        in_specs=[
            pl.BlockSpec((8, R_BLK, F0), lambda i: (0, i, 0)),
            hs_spec, hs_spec, hs_spec, hs_spec,
            pl.BlockSpec((8, R_BLK), lambda i: (0, i)),
            pl.BlockSpec((4, F0), lambda i: (0, 0)),
            pl.BlockSpec((F0, 2 * F1), lambda i: (0, 0)),
            pl.BlockSpec((F0, 2 * F1), lambda i: (0, 0)),
        ],
        out_specs=[pl.BlockSpec((R_BLK, F1), lambda i: (i, 0))] * 4,
        out_shape=[fo, fo, fo, fo],
    )(s0, hs_tt, hs_aa, hs_at, hs_ta, deg8, b0, w1tx, w1ad)


def _tc2_body(s_ref, hs_tt, hs_aa, hs_at, hs_ta, deg_ref, b1_ref,
              o_tx, o_ad):
    dinv_tt, dinv_aa, dinv_at, dinv_ta = _dinvs(deg_ref[...])
    s = s_ref[...]
    b1 = b1_ref[...]
    o_tx[...] = (dinv_tt[:, None] * (s[0] + s[1] + hs_tt[...])
                 + b1[0][None, :]
                 + dinv_at[:, None] * (s[4] + s[5] + hs_at[...])
                 + b1[2][None, :])
    o_ad[...] = (dinv_aa[:, None] * (s[2] + s[3] + hs_aa[...])
                 + b1[1][None, :]
                 + dinv_ta[:, None] * (s[6] + s[7] + hs_ta[...])
                 + b1[3][None, :])


def _tc2_call(s1, hs_tt, hs_aa, hs_at, hs_ta, deg8, b1):
    grid = NPAD // R_BLK
    fo = jax.ShapeDtypeStruct((NPAD, F1), jnp.float32)
    hs_spec = pl.BlockSpec((R_BLK, F1), lambda i: (i, 0))
    return pl.pallas_call(
        _tc2_body,
        grid=(grid,),
        in_specs=[
            pl.BlockSpec((8, R_BLK, F1), lambda i: (0, i, 0)),
            hs_spec, hs_spec, hs_spec, hs_spec,
            pl.BlockSpec((8, R_BLK), lambda i: (0, i)),
            pl.BlockSpec((4, F1), lambda i: (0, 0)),
        ],
        out_specs=[pl.BlockSpec((R_BLK, F1), lambda i: (i, 0))] * 2,
        out_shape=[fo, fo],
    )(s1, hs_tt, hs_aa, hs_at, hs_ta, deg8, b1)


def kernel(x_tx, x_addr, edge_index_tt, edge_index_aa, edge_index_at,
           edge_index_ta, W0_tt, b0_tt, W0_aa, b0_aa, W0_at, b0_at,
           W0_ta, b0_ta, W1_tt, b1_tt, W1_aa, b1_aa, W1_at, b1_at,
           W1_ta, b1_ta):
    xt = jnp.pad(x_tx, ((0, NPAD - N), (0, 0)))
    xa = jnp.pad(x_addr, ((0, NPAD - N), (0, 0)))

    def prep(ei):
        pad = jnp.full((EPAD - E,), N, jnp.int32)
        r = jnp.concatenate([ei[0], pad]).reshape(NTILES, QPT, CHUNK)
        c = jnp.concatenate([ei[1], pad]).reshape(NTILES, QPT, CHUNK)
        return r, c

    pairs = [prep(e) for e in
             (edge_index_tt, edge_index_aa, edge_index_at, edge_index_ta)]
    rows_all = jnp.stack([p[0] for p in pairs])
    cols_all = jnp.stack([p[1] for p in pairs])

    deg = _deg_call(cols_all)
    deg8 = deg.reshape(8, NPAD)

    w0tx = jnp.concatenate([W0_tt, W0_ta], axis=1)
    w0ad = jnp.concatenate([W0_aa, W0_at], axis=1)
    hs_tt, hs_aa, hs_at, hs_ta = _tc0_call(xt, xa, w0tx, w0ad, deg8)

    s0 = _scatter_call(F0, rows_all, cols_all, hs_tt, hs_aa, hs_at, hs_ta)

    b0 = jnp.stack([b0_tt, b0_aa, b0_at, b0_ta])
    w1tx = jnp.concatenate([W1_tt, W1_ta], axis=1)
    w1ad = jnp.concatenate([W1_aa, W1_at], axis=1)
    hs1_tt, hs1_aa, hs1_at, hs1_ta = _tc1_call(
        s0.reshape(8, NPAD, F0), hs_tt, hs_aa, hs_at, hs_ta,
        deg8, b0, w1tx, w1ad)

    s1 = _scatter_call(F1, rows_all, cols_all, hs1_tt, hs1_aa, hs1_at, hs1_ta)

    b1 = jnp.stack([b1_tt, b1_aa, b1_at, b1_ta])
    tx2, ad2 = _tc2_call(s1.reshape(8, NPAD, F1), hs1_tt, hs1_aa, hs1_at,
                         hs1_ta, deg8, b1)
    return tx2[:N], ad2[:N]


# speedup vs baseline: 14.9481x; 14.9481x over previous
import functools

import jax
import jax.numpy as jnp
from jax import lax
from jax.experimental import pallas as pl
from jax.experimental.pallas import tpu as pltpu
from jax.experimental.pallas import tpu_sc as plsc

N = 10000
NPAD = 10240
E = 160000
CHUNK = 128
NTILES = 32
QPT = 40
EPAD = NTILES * QPT * CHUNK
NC, NS = 2, 16
RPT = NPAD // NS
R_BLK = 1024
F0 = 64
F1 = 16


def _sc_mesh():
    return plsc.VectorSubcoreMesh(core_axis_name="c", subcore_axis_name="s")


def _deg_body(cols_hbm, out_hbm, accum, idxb, ones, zbuf):
    cid = lax.axis_index("c")
    sid = lax.axis_index("s")
    w = cid * NS + sid

    def fill_z(i, _):
        zbuf[pl.ds(i * 16, 16)] = jnp.zeros((16,), jnp.float32)
        return 0

    lax.fori_loop(0, RPT // 16, fill_z, 0)

    def fill_o(i, _):
        ones[pl.ds(i * 16, 16)] = jnp.ones((16,), jnp.float32)
        return 0

    lax.fori_loop(0, CHUNK // 16, fill_o, 0)

    for r in range(4):
        pltpu.sync_copy(zbuf, accum.at[pl.ds(sid * RPT, RPT)])
        plsc.subcore_barrier()
        pltpu.sync_copy(cols_hbm.at[r, w], idxb)

        def body(j, _):
            pltpu.sync_copy(ones, accum.at[idxb.at[j]], add=True)
            return 0

        lax.fori_loop(0, QPT, body, 0)
        plsc.subcore_barrier()
        pltpu.sync_copy(accum.at[pl.ds(sid * RPT, RPT)],
                        out_hbm.at[r, cid, pl.ds(sid * RPT, RPT)])
        plsc.subcore_barrier()


def _deg_call(cols_all):
    k = pl.kernel(
        _deg_body,
        out_type=jax.ShapeDtypeStruct((4, NC, NPAD), jnp.float32),
        mesh=_sc_mesh(),
        compiler_params=pltpu.CompilerParams(use_tc_tiling_on_sc=False),
        scratch_types=[
            pltpu.VMEM_SHARED((NPAD,), jnp.float32),
            pltpu.VMEM((QPT, CHUNK), jnp.int32),
            pltpu.VMEM((CHUNK,), jnp.float32),
            pltpu.VMEM((RPT,), jnp.float32),
        ],
    )
    return k(cols_all)


def _make_scatter_body(F):
    def body(rows_hbm, cols_hbm, t0, t1, t2, t3, out_hbm,
             accum, idxr, idxc, val, zb, sem):
        cid = lax.axis_index("c")
        sid = lax.axis_index("s")
        w = cid * NS + sid
        tables = [t0, t1, t2, t3]

        def fill_z(i, _):
            for kk in range(F // 16):
                zb[i, pl.ds(kk * 16, 16)] = jnp.zeros((16,), jnp.float32)
            return 0

        lax.fori_loop(0, CHUNK, fill_z, 0)

        for r in range(4):
            def zero_cp(j, _):
                pltpu.sync_copy(
                    zb, accum.at[pl.ds(sid * RPT + j * CHUNK, CHUNK)])
                return 0

            lax.fori_loop(0, RPT // CHUNK, zero_cp, 0)
            plsc.subcore_barrier()
            pltpu.sync_copy(rows_hbm.at[r, w], idxr)
            pltpu.sync_copy(cols_hbm.at[r, w], idxc)

            def chunk(j, _):
                pltpu.async_copy(tables[r].at[idxr.at[j]], val, sem).wait()
                pltpu.sync_copy(val, accum.at[idxc.at[j]], add=True)
                return 0

            lax.fori_loop(0, QPT, chunk, 0)
            plsc.subcore_barrier()
            pltpu.sync_copy(accum.at[pl.ds(sid * RPT, RPT)],
                            out_hbm.at[r, cid, pl.ds(sid * RPT, RPT)])
            plsc.subcore_barrier()

    return body


def _scatter_call(F, rows_all, cols_all, t_tt, t_aa, t_at, t_ta):
    k = pl.kernel(
        _make_scatter_body(F),
        out_type=jax.ShapeDtypeStruct((4, NC, NPAD, F), jnp.float32),
        mesh=_sc_mesh(),
        compiler_params=pltpu.CompilerParams(use_tc_tiling_on_sc=False),
        scratch_types=[
            pltpu.VMEM_SHARED((NPAD, F), jnp.float32),
            pltpu.VMEM((QPT, CHUNK), jnp.int32),
            pltpu.VMEM((QPT, CHUNK), jnp.int32),
            pltpu.VMEM((CHUNK, F), jnp.float32),
            pltpu.VMEM((CHUNK, F), jnp.float32),
            pltpu.SemaphoreType.DMA,
        ],
    )
    return k(rows_all, cols_all, t_tt, t_aa, t_at, t_ta)


def _dinvs(d):
    dinv_tt = lax.rsqrt(d[0] + d[1] + 1.0)
    dinv_aa = lax.rsqrt(d[2] + d[3] + 1.0)
    dinv_at = lax.rsqrt(d[4] + d[5] + 1.0)
    dinv_ta = lax.rsqrt(d[6] + d[7] + 1.0)
    return dinv_tt, dinv_aa, dinv_at, dinv_ta


def _tc0_body(xt_ref, xa_ref, wtx_ref, wad_ref, deg_ref,
              o_tt, o_aa, o_at, o_ta):
    dinv_tt, dinv_aa, dinv_at, dinv_ta = _dinvs(deg_ref[...])
    h_tx = jnp.dot(xt_ref[...], wtx_ref[...],
                   preferred_element_type=jnp.float32)
    h_ad = jnp.dot(xa_ref[...], wad_ref[...],
                   preferred_element_type=jnp.float32)
    o_tt[...] = h_tx[:, :F0] * dinv_tt[:, None]
    o_ta[...] = h_tx[:, F0:] * dinv_ta[:, None]
    o_aa[...] = h_ad[:, :F0] * dinv_aa[:, None]
    o_at[...] = h_ad[:, F0:] * dinv_at[:, None]


def _tc0_call(xt, xa, wtx, wad, deg8):
    grid = NPAD // R_BLK
    fo = jax.ShapeDtypeStruct((NPAD, F0), jnp.float32)
    return pl.pallas_call(
        _tc0_body,
        grid=(grid,),
        in_specs=[
            pl.BlockSpec((R_BLK, 128), lambda i: (i, 0)),
            pl.BlockSpec((R_BLK, 128), lambda i: (i, 0)),
            pl.BlockSpec((128, 128), lambda i: (0, 0)),
            pl.BlockSpec((128, 128), lambda i: (0, 0)),
            pl.BlockSpec((8, R_BLK), lambda i: (0, i)),
        ],
        out_specs=[pl.BlockSpec((R_BLK, F0), lambda i: (i, 0))] * 4,
        out_shape=[fo, fo, fo, fo],
    )(xt, xa, wtx, wad, deg8)


def _tc1_body(s_ref, hs_tt, hs_aa, hs_at, hs_ta, deg_ref, b0_ref,
              w1tx_ref, w1ad_ref, o_tt, o_aa, o_at, o_ta):
    dinv_tt, dinv_aa, dinv_at, dinv_ta = _dinvs(deg_ref[...])
    s = s_ref[...]
    b0 = b0_ref[...]
    tx1 = (dinv_tt[:, None] * (s[0] + s[1] + hs_tt[...]) + b0[0][None, :]
           + dinv_at[:, None] * (s[4] + s[5] + hs_at[...]) + b0[2][None, :])
    ad1 = (dinv_aa[:, None] * (s[2] + s[3] + hs_aa[...]) + b0[1][None, :]
           + dinv_ta[:, None] * (s[6] + s[7] + hs_ta[...]) + b0[3][None, :])
    tx1 = jnp.maximum(tx1, 0.0)
    ad1 = jnp.maximum(ad1, 0.0)
    h1tx = jnp.dot(tx1, w1tx_ref[...], preferred_element_type=jnp.float32)
    h1ad = jnp.dot(ad1, w1ad_ref[...], preferred_element_type=jnp.float32)
    o_tt[...] = h1tx[:, :F1] * dinv_tt[:, None]
    o_ta[...] = h1tx[:, F1:] * dinv_ta[:, None]
    o_aa[...] = h1ad[:, :F1] * dinv_aa[:, None]
    o_at[...] = h1ad[:, F1:] * dinv_at[:, None]


def _tc1_call(s0, hs_tt, hs_aa, hs_at, hs_ta, deg8, b0, w1tx, w1ad):
    grid = NPAD // R_BLK
    fo = jax.ShapeDtypeStruct((NPAD, F1), jnp.float32)
    hs_spec = pl.BlockSpec((R_BLK, F0), lambda i: (i, 0))
    return pl.pallas_call(
        _tc1_body,
        grid=(grid,),
        in_specs=[
            pl.BlockSpec((8, R_BLK, F0), lambda i: (0, i, 0)),
            hs_spec, hs_spec, hs_spec, hs_spec,
            pl.BlockSpec((8, R_BLK), lambda i: (0, i)),
            pl.BlockSpec((4, F0), lambda i: (0, 0)),
            pl.BlockSpec((F0, 2 * F1), lambda i: (0, 0)),
            pl.BlockSpec((F0, 2 * F1), lambda i: (0, 0)),
        ],
        out_specs=[pl.BlockSpec((R_BLK, F1), lambda i: (i, 0))] * 4,
        out_shape=[fo, fo, fo, fo],
    )(s0, hs_tt, hs_aa, hs_at, hs_ta, deg8, b0, w1tx, w1ad)


def _tc2_body(s_ref, hs_tt, hs_aa, hs_at, hs_ta, deg_ref, b1_ref,
              o_tx, o_ad):
    dinv_tt, dinv_aa, dinv_at, dinv_ta = _dinvs(deg_ref[...])
    s = s_ref[...]
    b1 = b1_ref[...]
    o_tx[...] = (dinv_tt[:, None] * (s[0] + s[1] + hs_tt[...])
                 + b1[0][None, :]
                 + dinv_at[:, None] * (s[4] + s[5] + hs_at[...])
                 + b1[2][None, :])
    o_ad[...] = (dinv_aa[:, None] * (s[2] + s[3] + hs_aa[...])
                 + b1[1][None, :]
                 + dinv_ta[:, None] * (s[6] + s[7] + hs_ta[...])
                 + b1[3][None, :])


def _tc2_call(s1, hs_tt, hs_aa, hs_at, hs_ta, deg8, b1):
    grid = NPAD // R_BLK
    fo = jax.ShapeDtypeStruct((NPAD, F1), jnp.float32)
    hs_spec = pl.BlockSpec((R_BLK, F1), lambda i: (i, 0))
    return pl.pallas_call(
        _tc2_body,
        grid=(grid,),
        in_specs=[
            pl.BlockSpec((8, R_BLK, F1), lambda i: (0, i, 0)),
            hs_spec, hs_spec, hs_spec, hs_spec,
            pl.BlockSpec((8, R_BLK), lambda i: (0, i)),
            pl.BlockSpec((4, F1), lambda i: (0, 0)),
        ],
        out_specs=[pl.BlockSpec((R_BLK, F1), lambda i: (i, 0))] * 2,
        out_shape=[fo, fo],
    )(s1, hs_tt, hs_aa, hs_at, hs_ta, deg8, b1)


def kernel(x_tx, x_addr, edge_index_tt, edge_index_aa, edge_index_at,
           edge_index_ta, W0_tt, b0_tt, W0_aa, b0_aa, W0_at, b0_at,
           W0_ta, b0_ta, W1_tt, b1_tt, W1_aa, b1_aa, W1_at, b1_at,
           W1_ta, b1_ta):
    xt = jnp.pad(x_tx, ((0, NPAD - N), (0, 0)))
    xa = jnp.pad(x_addr, ((0, NPAD - N), (0, 0)))

    def prep(ei):
        pad = jnp.full((EPAD - E,), N, jnp.int32)
        r = jnp.concatenate([ei[0], pad]).reshape(NTILES, QPT, CHUNK)
        c = jnp.concatenate([ei[1], pad]).reshape(NTILES, QPT, CHUNK)
        return r, c

    pairs = [prep(e) for e in
             (edge_index_tt, edge_index_aa, edge_index_at, edge_index_ta)]
    rows_all = jnp.stack([p[0] for p in pairs])
    cols_all = jnp.stack([p[1] for p in pairs])

    deg = _deg_call(cols_all)
    deg8 = deg.reshape(8, NPAD)

    w0tx = jnp.concatenate([W0_tt, W0_ta], axis=1)
    w0ad = jnp.concatenate([W0_aa, W0_at], axis=1)
    hs_tt, hs_aa, hs_at, hs_ta = _tc0_call(xt, xa, w0tx, w0ad, deg8)

    s0 = _scatter_call(F0, rows_all, cols_all, hs_tt, hs_aa, hs_at, hs_ta)

    b0 = jnp.stack([b0_tt, b0_aa, b0_at, b0_ta])
    w1tx = jnp.concatenate([W1_tt, W1_ta], axis=1)
    w1ad = jnp.concatenate([W1_aa, W1_at], axis=1)
    hs1_tt, hs1_aa, hs1_at, hs1_ta = _tc1_call(
        s0.reshape(8, NPAD, F0), hs_tt, hs_aa, hs_at, hs_ta,
        deg8, b0, w1tx, w1ad)

    s1 = _scatter_call(F1, rows_all, cols_all, hs1_tt, hs1_aa, hs1_at, hs1_ta)

    b1 = jnp.stack([b1_tt, b1_aa, b1_at, b1_ta])
    tx2, ad2 = _tc2_call(s1.reshape(8, NPAD, F1), hs1_tt, hs1_aa, hs1_at,
                         hs1_ta, deg8, b1)
    return tx2[:N], ad2[:N]


# speedup vs baseline: 31.9578x; 2.1379x over previous
import functools

import jax
import jax.numpy as jnp
from jax import lax
from jax.experimental import pallas as pl
from jax.experimental.pallas import tpu as pltpu
from jax.experimental.pallas import tpu_sc as plsc

N = 10000
NPAD = 10240
E = 160000
CHUNK = 128
NTILES = 32
QPT = 40
EPAD = NTILES * QPT * CHUNK
NC, NS = 2, 16
RPT = NPAD // NS
R_BLK = 1024
F0 = 64
F1 = 16


def _sc_mesh():
    return plsc.VectorSubcoreMesh(core_axis_name="c", subcore_axis_name="s")


def _deg_body(cols_hbm, out_hbm, accum, idxb, ones, zbuf):
    cid = lax.axis_index("c")
    sid = lax.axis_index("s")
    w = cid * NS + sid

    def fill_z(i, _):
        zbuf[pl.ds(i * 16, 16)] = jnp.zeros((16,), jnp.float32)
        return 0

    lax.fori_loop(0, RPT // 16, fill_z, 0)

    def fill_o(i, _):
        ones[pl.ds(i * 16, 16)] = jnp.ones((16,), jnp.float32)
        return 0

    lax.fori_loop(0, CHUNK // 16, fill_o, 0)

    for r in range(4):
        pltpu.sync_copy(zbuf, accum.at[pl.ds(sid * RPT, RPT)])
        plsc.subcore_barrier()
        pltpu.sync_copy(cols_hbm.at[r, w], idxb)

        def body(j, _):
            pltpu.sync_copy(ones, accum.at[idxb.at[j]], add=True)
            return 0

        lax.fori_loop(0, QPT, body, 0)
        plsc.subcore_barrier()
        pltpu.sync_copy(accum.at[pl.ds(sid * RPT, RPT)],
                        out_hbm.at[r, cid, pl.ds(sid * RPT, RPT)])
        plsc.subcore_barrier()


def _deg_call(cols_all):
    k = pl.kernel(
        _deg_body,
        out_type=jax.ShapeDtypeStruct((4, NC, NPAD), jnp.float32),
        mesh=_sc_mesh(),
        compiler_params=pltpu.CompilerParams(use_tc_tiling_on_sc=False),
        scratch_types=[
            pltpu.VMEM_SHARED((NPAD,), jnp.float32),
            pltpu.VMEM((QPT, CHUNK), jnp.int32),
            pltpu.VMEM((CHUNK,), jnp.float32),
            pltpu.VMEM((RPT,), jnp.float32),
        ],
    )
    return k(cols_all)


def _make_scatter_body(F, NBUF):
    def body(rows_hbm, cols_hbm, t0, t1, t2, t3, out_hbm,
             accum, table_sh, idxr, idxc, vals, zb, gsems):
        cid = lax.axis_index("c")
        sid = lax.axis_index("s")
        w = cid * NS + sid
        tables = [t0, t1, t2, t3]

        def fill_z(i, _):
            for kk in range(F // 16):
                zb[i, pl.ds(kk * 16, 16)] = jnp.zeros((16,), jnp.float32)
            return 0

        lax.fori_loop(0, CHUNK, fill_z, 0)

        def fire(cur, b):
            return pltpu.async_copy(
                table_sh.at[idxr.at[cur]], vals[b], gsems[b])

        def wait(cur, b):
            pltpu.make_async_copy(
                table_sh.at[idxr.at[cur]], vals[b], gsems[b]).wait()

        for r in range(4):
            def zero_cp(j, _):
                pltpu.sync_copy(
                    zb, accum.at[pl.ds(sid * RPT + j * CHUNK, CHUNK)])
                return 0

            lax.fori_loop(0, RPT // CHUNK, zero_cp, 0)
            pltpu.sync_copy(rows_hbm.at[r, w], idxr)
            pltpu.sync_copy(cols_hbm.at[r, w], idxc)
            pltpu.sync_copy(tables[r].at[pl.ds(sid * RPT, RPT)],
                            table_sh.at[pl.ds(sid * RPT, RPT)])
            plsc.subcore_barrier()

            for b in range(NBUF):
                fire(b, b)

            def chunk(j, _):
                for b in range(NBUF):
                    cur = j * NBUF + b
                    wait(cur, b)
                    pltpu.sync_copy(vals[b], accum.at[idxc.at[cur]],
                                    add=True)
                    fire(cur + NBUF, b)
                return 0

            lax.fori_loop(0, QPT // NBUF - 1, chunk, 0)
            for b in range(NBUF):
                cur = QPT - NBUF + b
                wait(cur, b)
                pltpu.sync_copy(vals[b], accum.at[idxc.at[cur]], add=True)
            plsc.subcore_barrier()
            pltpu.sync_copy(accum.at[pl.ds(sid * RPT, RPT)],
                            out_hbm.at[r, cid, pl.ds(sid * RPT, RPT)])
            plsc.subcore_barrier()

    return body


def _scatter_call(F, NBUF, rows_all, cols_all, t_tt, t_aa, t_at, t_ta):
    k = pl.kernel(
        _make_scatter_body(F, NBUF),
        out_type=jax.ShapeDtypeStruct((4, NC, NPAD, F), jnp.float32),
        mesh=_sc_mesh(),
        compiler_params=pltpu.CompilerParams(use_tc_tiling_on_sc=False),
        scratch_types=[
            pltpu.VMEM_SHARED((NPAD, F), jnp.float32),
            pltpu.VMEM_SHARED((NPAD, F), jnp.float32),
            pltpu.VMEM((QPT, CHUNK), jnp.int32),
            pltpu.VMEM((QPT, CHUNK), jnp.int32),
            [pltpu.VMEM((CHUNK, F), jnp.float32)] * NBUF,
            pltpu.VMEM((CHUNK, F), jnp.float32),
            [pltpu.SemaphoreType.DMA] * NBUF,
        ],
    )
    return k(rows_all, cols_all, t_tt, t_aa, t_at, t_ta)


def _dinvs(d):
    dinv_tt = lax.rsqrt(d[0] + d[1] + 1.0)
    dinv_aa = lax.rsqrt(d[2] + d[3] + 1.0)
    dinv_at = lax.rsqrt(d[4] + d[5] + 1.0)
    dinv_ta = lax.rsqrt(d[6] + d[7] + 1.0)
    return dinv_tt, dinv_aa, dinv_at, dinv_ta


def _tc0_body(xt_ref, xa_ref, wtx_ref, wad_ref, deg_ref,
              o_tt, o_aa, o_at, o_ta):
    dinv_tt, dinv_aa, dinv_at, dinv_ta = _dinvs(deg_ref[...])
    h_tx = jnp.dot(xt_ref[...], wtx_ref[...],
                   preferred_element_type=jnp.float32)
    h_ad = jnp.dot(xa_ref[...], wad_ref[...],
                   preferred_element_type=jnp.float32)
    o_tt[...] = h_tx[:, :F0] * dinv_tt[:, None]
    o_ta[...] = h_tx[:, F0:] * dinv_ta[:, None]
    o_aa[...] = h_ad[:, :F0] * dinv_aa[:, None]
    o_at[...] = h_ad[:, F0:] * dinv_at[:, None]


def _tc0_call(xt, xa, wtx, wad, deg8):
    grid = NPAD // R_BLK
    fo = jax.ShapeDtypeStruct((NPAD, F0), jnp.float32)
    return pl.pallas_call(
        _tc0_body,
        grid=(grid,),
        in_specs=[
            pl.BlockSpec((R_BLK, 128), lambda i: (i, 0)),
            pl.BlockSpec((R_BLK, 128), lambda i: (i, 0)),
            pl.BlockSpec((128, 128), lambda i: (0, 0)),
            pl.BlockSpec((128, 128), lambda i: (0, 0)),
            pl.BlockSpec((8, R_BLK), lambda i: (0, i)),
        ],
        out_specs=[pl.BlockSpec((R_BLK, F0), lambda i: (i, 0))] * 4,
        out_shape=[fo, fo, fo, fo],
    )(xt, xa, wtx, wad, deg8)


def _tc1_body(s_ref, hs_tt, hs_aa, hs_at, hs_ta, deg_ref, b0_ref,
              w1tx_ref, w1ad_ref, o_tt, o_aa, o_at, o_ta):
    dinv_tt, dinv_aa, dinv_at, dinv_ta = _dinvs(deg_ref[...])
    s = s_ref[...]
    b0 = b0_ref[...]
    tx1 = (dinv_tt[:, None] * (s[0] + s[1] + hs_tt[...]) + b0[0][None, :]
           + dinv_at[:, None] * (s[4] + s[5] + hs_at[...]) + b0[2][None, :])
    ad1 = (dinv_aa[:, None] * (s[2] + s[3] + hs_aa[...]) + b0[1][None, :]
           + dinv_ta[:, None] * (s[6] + s[7] + hs_ta[...]) + b0[3][None, :])
    tx1 = jnp.maximum(tx1, 0.0)
    ad1 = jnp.maximum(ad1, 0.0)
    h1tx = jnp.dot(tx1, w1tx_ref[...], preferred_element_type=jnp.float32)
    h1ad = jnp.dot(ad1, w1ad_ref[...], preferred_element_type=jnp.float32)
    o_tt[...] = h1tx[:, :F1] * dinv_tt[:, None]
    o_ta[...] = h1tx[:, F1:] * dinv_ta[:, None]
    o_aa[...] = h1ad[:, :F1] * dinv_aa[:, None]
    o_at[...] = h1ad[:, F1:] * dinv_at[:, None]


def _tc1_call(s0, hs_tt, hs_aa, hs_at, hs_ta, deg8, b0, w1tx, w1ad):
    grid = NPAD // R_BLK
    fo = jax.ShapeDtypeStruct((NPAD, F1), jnp.float32)
    hs_spec = pl.BlockSpec((R_BLK, F0), lambda i: (i, 0))
    return pl.pallas_call(
        _tc1_body,
        grid=(grid,),
        in_specs=[
            pl.BlockSpec((8, R_BLK, F0), lambda i: (0, i, 0)),
            hs_spec, hs_spec, hs_spec, hs_spec,
            pl.BlockSpec((8, R_BLK), lambda i: (0, i)),
            pl.BlockSpec((4, F0), lambda i: (0, 0)),
            pl.BlockSpec((F0, 2 * F1), lambda i: (0, 0)),
            pl.BlockSpec((F0, 2 * F1), lambda i: (0, 0)),
        ],
        out_specs=[pl.BlockSpec((R_BLK, F1), lambda i: (i, 0))] * 4,
        out_shape=[fo, fo, fo, fo],
    )(s0, hs_tt, hs_aa, hs_at, hs_ta, deg8, b0, w1tx, w1ad)


def _tc2_body(s_ref, hs_tt, hs_aa, hs_at, hs_ta, deg_ref, b1_ref,
              o_tx, o_ad):
    dinv_tt, dinv_aa, dinv_at, dinv_ta = _dinvs(deg_ref[...])
    s = s_ref[...]
    b1 = b1_ref[...]
    o_tx[...] = (dinv_tt[:, None] * (s[0] + s[1] + hs_tt[...])
                 + b1[0][None, :]
                 + dinv_at[:, None] * (s[4] + s[5] + hs_at[...])
                 + b1[2][None, :])
    o_ad[...] = (dinv_aa[:, None] * (s[2] + s[3] + hs_aa[...])
                 + b1[1][None, :]
                 + dinv_ta[:, None] * (s[6] + s[7] + hs_ta[...])
                 + b1[3][None, :])


def _tc2_call(s1, hs_tt, hs_aa, hs_at, hs_ta, deg8, b1):
    grid = NPAD // R_BLK
    fo = jax.ShapeDtypeStruct((NPAD, F1), jnp.float32)
    hs_spec = pl.BlockSpec((R_BLK, F1), lambda i: (i, 0))
    return pl.pallas_call(
        _tc2_body,
        grid=(grid,),
        in_specs=[
            pl.BlockSpec((8, R_BLK, F1), lambda i: (0, i, 0)),
            hs_spec, hs_spec, hs_spec, hs_spec,
            pl.BlockSpec((8, R_BLK), lambda i: (0, i)),
            pl.BlockSpec((4, F1), lambda i: (0, 0)),
        ],
        out_specs=[pl.BlockSpec((R_BLK, F1), lambda i: (i, 0))] * 2,
        out_shape=[fo, fo],
    )(s1, hs_tt, hs_aa, hs_at, hs_ta, deg8, b1)


def kernel(x_tx, x_addr, edge_index_tt, edge_index_aa, edge_index_at,
           edge_index_ta, W0_tt, b0_tt, W0_aa, b0_aa, W0_at, b0_at,
           W0_ta, b0_ta, W1_tt, b1_tt, W1_aa, b1_aa, W1_at, b1_at,
           W1_ta, b1_ta):
    xt = jnp.pad(x_tx, ((0, NPAD - N), (0, 0)))
    xa = jnp.pad(x_addr, ((0, NPAD - N), (0, 0)))

    def prep(ei):
        pad = jnp.full((EPAD - E,), N, jnp.int32)
        r = jnp.concatenate([ei[0], pad]).reshape(NTILES, QPT, CHUNK)
        c = jnp.concatenate([ei[1], pad]).reshape(NTILES, QPT, CHUNK)
        return r, c

    pairs = [prep(e) for e in
             (edge_index_tt, edge_index_aa, edge_index_at, edge_index_ta)]
    rows_all = jnp.stack([p[0] for p in pairs])
    cols_all = jnp.stack([p[1] for p in pairs])

    deg = _deg_call(cols_all)
    deg8 = deg.reshape(8, NPAD)

    w0tx = jnp.concatenate([W0_tt, W0_ta], axis=1)
    w0ad = jnp.concatenate([W0_aa, W0_at], axis=1)
    hs_tt, hs_aa, hs_at, hs_ta = _tc0_call(xt, xa, w0tx, w0ad, deg8)

    s0 = _scatter_call(F0, 2, rows_all, cols_all, hs_tt, hs_aa, hs_at, hs_ta)

    b0 = jnp.stack([b0_tt, b0_aa, b0_at, b0_ta])
    w1tx = jnp.concatenate([W1_tt, W1_ta], axis=1)
    w1ad = jnp.concatenate([W1_aa, W1_at], axis=1)
    hs1_tt, hs1_aa, hs1_at, hs1_ta = _tc1_call(
        s0.reshape(8, NPAD, F0), hs_tt, hs_aa, hs_at, hs_ta,
        deg8, b0, w1tx, w1ad)

    s1 = _scatter_call(F1, 4, rows_all, cols_all,
                       hs1_tt, hs1_aa, hs1_at, hs1_ta)

    b1 = jnp.stack([b1_tt, b1_aa, b1_at, b1_ta])
    tx2, ad2 = _tc2_call(s1.reshape(8, NPAD, F1), hs1_tt, hs1_aa, hs1_at,
                         hs1_ta, deg8, b1)
    return tx2[:N], ad2[:N]
